# initial kernel scaffold (unmeasured)
import jax
import jax.numpy as jnp
from jax import lax
from jax.experimental import pallas as pl
from jax.experimental.pallas import tpu as pltpu

N_DEV = 8
FP8 = jnp.float8_e4m3fn


def kernel(x, w_mat, scale_x, scale_w):
    m_glob, k_loc = x.shape
    k_glob, n = w_mat.shape
    m_loc = m_glob // N_DEV

    def body(x_ref, w_ref, sx_ref, sw_ref, out_ref,
             send_buf, recv_buf, send_sems, recv_sems):
        my = lax.axis_index("i")

        barrier = pltpu.get_barrier_semaphore()
        for k in range(1, N_DEV):
            peer = lax.rem(my + k, N_DEV)
            pl.semaphore_signal(
                barrier, inc=1,
                device_id=(peer,), device_id_type=pl.DeviceIdType.MESH,
            )
        pl.semaphore_wait(barrier, N_DEV - 1)

        for j in range(N_DEV):
            send_buf[j] = x_ref[j * m_loc:(j + 1) * m_loc, :].astype(FP8)
        recv_buf[my] = send_buf[my]

        sends = []
        for k in range(1, N_DEV):
            peer = lax.rem(my + k, N_DEV)
            rdma = pltpu.make_async_remote_copy(
                src_ref=send_buf.at[peer],
                dst_ref=recv_buf.at[my],
                send_sem=send_sems.at[peer],
                recv_sem=recv_sems.at[my],
                device_id=(peer,),
                device_id_type=pl.DeviceIdType.MESH,
            )
            rdma.start()
            sends.append(rdma)

        for k in range(1, N_DEV):
            src = lax.rem(my + k, N_DEV)
            recv = pltpu.make_async_remote_copy(
                src_ref=send_buf.at[src],
                dst_ref=recv_buf.at[src],
                send_sem=send_sems.at[src],
                recv_sem=recv_sems.at[src],
                device_id=(src,),
                device_id_type=pl.DeviceIdType.MESH,
            )
            recv.wait_recv()

        acc = jnp.zeros((m_loc, n), jnp.float32)
        for s in range(N_DEV):
            a_s = recv_buf[s].astype(jnp.float32)
            acc = acc + jnp.dot(
                a_s, w_ref[s * k_loc:(s + 1) * k_loc, :],
                preferred_element_type=jnp.float32,
            )

        y = acc * (sx_ref[0] * sw_ref[0])
        out_ref[...] = y * jax.nn.sigmoid(y)

        for rdma in sends:
            rdma.wait_send()

    return pl.pallas_call(
        body,
        out_shape=jax.ShapeDtypeStruct((m_loc, n), jnp.float32),
        in_specs=[
            pl.BlockSpec(memory_space=pltpu.VMEM),
            pl.BlockSpec(memory_space=pltpu.VMEM),
            pl.BlockSpec(memory_space=pltpu.SMEM),
            pl.BlockSpec(memory_space=pltpu.SMEM),
        ],
        out_specs=pl.BlockSpec(memory_space=pltpu.VMEM),
        scratch_shapes=[
            pltpu.VMEM((N_DEV, m_loc, k_loc), FP8),
            pltpu.VMEM((N_DEV, m_loc, k_loc), FP8),
            pltpu.SemaphoreType.DMA((N_DEV,)),
            pltpu.SemaphoreType.DMA((N_DEV,)),
        ],
        compiler_params=pltpu.CompilerParams(collective_id=0),
    )(x, w_mat, scale_x, scale_w)


# baseline (device time: 50550 ns/iter reference)
import jax
import jax.numpy as jnp
from jax import lax
from jax.experimental import pallas as pl
from jax.experimental.pallas import tpu as pltpu

N_DEV = 8
FP8 = jnp.float8_e4m3fn


def kernel(x, w_mat, scale_x, scale_w):
    m_glob, k_loc = x.shape
    k_glob, n = w_mat.shape
    m_loc = m_glob // N_DEV

    def body(x_ref, w_ref, sx_ref, sw_ref, out_ref,
             send_buf, recv_buf, send_sems, recv_sems):
        my = lax.axis_index("i")

        barrier = pltpu.get_barrier_semaphore()
        for k in range(1, N_DEV):
            peer = lax.rem(my + k, N_DEV)
            pl.semaphore_signal(
                barrier, inc=1,
                device_id=(peer,), device_id_type=pl.DeviceIdType.MESH,
            )
        pl.semaphore_wait(barrier, N_DEV - 1)

        for j in range(N_DEV):
            send_buf[j] = x_ref[j * m_loc:(j + 1) * m_loc, :].astype(FP8)
        recv_buf[my] = send_buf[my]

        sends = []
        for k in range(1, N_DEV):
            peer = lax.rem(my + k, N_DEV)
            rdma = pltpu.make_async_remote_copy(
                src_ref=send_buf.at[peer],
                dst_ref=recv_buf.at[my],
                send_sem=send_sems.at[peer],
                recv_sem=recv_sems.at[my],
                device_id=(peer,),
                device_id_type=pl.DeviceIdType.MESH,
            )
            rdma.start()
            sends.append(rdma)

        for k in range(1, N_DEV):
            src = lax.rem(my + k, N_DEV)
            recv = pltpu.make_async_remote_copy(
                src_ref=send_buf.at[src],
                dst_ref=recv_buf.at[src],
                send_sem=send_sems.at[src],
                recv_sem=recv_sems.at[src],
                device_id=(src,),
                device_id_type=pl.DeviceIdType.MESH,
            )
            recv.wait_recv()

        acc = jnp.zeros((m_loc, n), jnp.float32)
        for s in range(N_DEV):
            a_s = recv_buf[s].astype(jnp.float32)
            acc = acc + jnp.dot(
                a_s, w_ref[s * k_loc:(s + 1) * k_loc, :],
                preferred_element_type=jnp.float32,
            )

        y = acc * (sx_ref[0] * sw_ref[0])
        out_ref[...] = y * jax.nn.sigmoid(y)

        for rdma in sends:
            rdma.wait_send()

    return pl.pallas_call(
        body,
        out_shape=jax.ShapeDtypeStruct((m_loc, n), jnp.float32),
        in_specs=[
            pl.BlockSpec(memory_space=pltpu.VMEM),
            pl.BlockSpec(memory_space=pltpu.VMEM),
            pl.BlockSpec(memory_space=pltpu.SMEM),
            pl.BlockSpec(memory_space=pltpu.SMEM),
        ],
        out_specs=pl.BlockSpec(memory_space=pltpu.VMEM),
        scratch_shapes=[
            pltpu.VMEM((N_DEV, m_loc, k_loc), FP8),
            pltpu.VMEM((N_DEV, m_loc, k_loc), FP8),
            pltpu.SemaphoreType.DMA((N_DEV,)),
            pltpu.SemaphoreType.DMA((N_DEV,)),
        ],
        compiler_params=pltpu.CompilerParams(
            collective_id=0,
            vmem_limit_bytes=63 * 1024 * 1024,
        ),
    )(x, w_mat, scale_x, scale_w)


# device time: 38863 ns/iter; 1.3007x vs baseline; 1.3007x over previous
import jax
import jax.numpy as jnp
from jax import lax
from jax.experimental import pallas as pl
from jax.experimental.pallas import tpu as pltpu

N_DEV = 8
FP8 = jnp.float8_e4m3fn
W_SLOTS = 4


def kernel(x, w_mat, scale_x, scale_w):
    m_glob, k_loc = x.shape
    k_glob, n = w_mat.shape
    m_loc = m_glob // N_DEV

    def body(x_hbm, w_hbm, sx_ref, sw_ref, out_ref,
             x_vmem, w_vmem, send_buf, recv_buf,
             x_sems, w_sems, send_sems, recv_sems):
        my = lax.axis_index("i")
        srcs = [lax.rem(my + i, N_DEV) for i in range(N_DEV)]

        x_order = srcs[1:] + [my]
        x_copies = []
        for i, blk in enumerate(x_order):
            cp = pltpu.make_async_copy(
                x_hbm.at[pl.ds(blk * m_loc, m_loc), :],
                x_vmem.at[i],
                x_sems.at[i],
            )
            cp.start()
            x_copies.append(cp)

        def start_w(i):
            cp = pltpu.make_async_copy(
                w_hbm.at[pl.ds(srcs[i] * k_loc, k_loc), :],
                w_vmem.at[i % W_SLOTS],
                w_sems.at[i % W_SLOTS],
            )
            cp.start()
            return cp

        w_copies = {i: start_w(i) for i in range(W_SLOTS)}

        barrier = pltpu.get_barrier_semaphore()
        for k in range(1, N_DEV):
            pl.semaphore_signal(
                barrier, inc=1,
                device_id=(srcs[k],), device_id_type=pl.DeviceIdType.MESH,
            )
        pl.semaphore_wait(barrier, N_DEV - 1)

        sends = []
        for k in range(1, N_DEV):
            peer = srcs[k]
            x_copies[k - 1].wait()
            send_buf[k - 1] = x_vmem[k - 1].astype(FP8)
            rdma = pltpu.make_async_remote_copy(
                src_ref=send_buf.at[k - 1],
                dst_ref=recv_buf.at[my],
                send_sem=send_sems.at[k - 1],
                recv_sem=recv_sems.at[my],
                device_id=(peer,),
                device_id_type=pl.DeviceIdType.MESH,
            )
            rdma.start()
            sends.append(rdma)
        x_copies[N_DEV - 1].wait()
        recv_buf[my] = x_vmem[N_DEV - 1].astype(FP8)

        acc = jnp.zeros((m_loc, n), jnp.float32)
        for i in range(N_DEV):
            if i > 0:
                recv = pltpu.make_async_remote_copy(
                    src_ref=send_buf.at[0],
                    dst_ref=recv_buf.at[srcs[i]],
                    send_sem=send_sems.at[0],
                    recv_sem=recv_sems.at[srcs[i]],
                    device_id=(srcs[i],),
                    device_id_type=pl.DeviceIdType.MESH,
                )
                recv.wait_recv()
            w_copies[i].wait()
            a_i = recv_buf[srcs[i]].astype(jnp.float32)
            acc = acc + jnp.dot(
                a_i, w_vmem[i % W_SLOTS],
                preferred_element_type=jnp.float32,
            )
            if i + W_SLOTS < N_DEV:
                w_copies[i + W_SLOTS] = start_w(i + W_SLOTS)

        y = acc * (sx_ref[0] * sw_ref[0])
        out_ref[...] = y * jax.nn.sigmoid(y)

        for rdma in sends:
            rdma.wait_send()

    return pl.pallas_call(
        body,
        out_shape=jax.ShapeDtypeStruct((m_loc, n), jnp.float32),
        in_specs=[
            pl.BlockSpec(memory_space=pl.ANY),
            pl.BlockSpec(memory_space=pl.ANY),
            pl.BlockSpec(memory_space=pltpu.SMEM),
            pl.BlockSpec(memory_space=pltpu.SMEM),
        ],
        out_specs=pl.BlockSpec(memory_space=pltpu.VMEM),
        scratch_shapes=[
            pltpu.VMEM((N_DEV, m_loc, k_loc), jnp.float32),
            pltpu.VMEM((W_SLOTS, k_loc, n), jnp.float32),
            pltpu.VMEM((N_DEV - 1, m_loc, k_loc), FP8),
            pltpu.VMEM((N_DEV, m_loc, k_loc), FP8),
            pltpu.SemaphoreType.DMA((N_DEV,)),
            pltpu.SemaphoreType.DMA((W_SLOTS,)),
            pltpu.SemaphoreType.DMA((N_DEV - 1,)),
            pltpu.SemaphoreType.DMA((N_DEV,)),
        ],
        compiler_params=pltpu.CompilerParams(
            collective_id=0,
            vmem_limit_bytes=63 * 1024 * 1024,
        ),
    )(x, w_mat, scale_x, scale_w)


# device time: 30396 ns/iter; 1.6630x vs baseline; 1.2786x over previous
import jax
import jax.numpy as jnp
from jax import lax
from jax.experimental import pallas as pl
from jax.experimental.pallas import tpu as pltpu

N_DEV = 8
FP8 = jnp.float8_e4m3fn
W_SLOTS = 4


def kernel(x, w_mat, scale_x, scale_w):
    m_glob, k_loc = x.shape
    k_glob, n = w_mat.shape
    m_loc = m_glob // N_DEV

    def body(x_hbm, w_hbm, sx_ref, sw_ref, out_hbm,
             x_vmem, w_vmem, send_buf, recv_buf, y_vmem,
             x_sems, w_sems, send_sems, recv_sems, out_sems):
        my = lax.axis_index("i")
        send_peers = [lax.rem(my + k, N_DEV) for k in range(1, N_DEV)]
        srcs = [my] + [lax.rem(my + N_DEV - i, N_DEV) for i in range(1, N_DEV)]

        x_order = send_peers + [my]
        x_copies = []
        for i, blk in enumerate(x_order):
            cp = pltpu.make_async_copy(
                x_hbm.at[pl.ds(blk * m_loc, m_loc), :],
                x_vmem.at[i],
                x_sems.at[i],
            )
            cp.start()
            x_copies.append(cp)

        def start_w(i):
            cp = pltpu.make_async_copy(
                w_hbm.at[pl.ds(srcs[i] * k_loc, k_loc), :],
                w_vmem.at[i % W_SLOTS],
                w_sems.at[i % W_SLOTS],
            )
            cp.start()
            return cp

        w_copies = {i: start_w(i) for i in range(W_SLOTS)}

        barrier = pltpu.get_barrier_semaphore()
        for peer in send_peers:
            pl.semaphore_signal(
                barrier, inc=1,
                device_id=(peer,), device_id_type=pl.DeviceIdType.MESH,
            )
        pl.semaphore_wait(barrier, N_DEV - 1)

        sends = []
        for k in range(1, N_DEV):
            peer = send_peers[k - 1]
            x_copies[k - 1].wait()
            send_buf[k - 1] = x_vmem[k - 1].astype(FP8)
            rdma = pltpu.make_async_remote_copy(
                src_ref=send_buf.at[k - 1],
                dst_ref=recv_buf.at[my],
                send_sem=send_sems.at[k - 1],
                recv_sem=recv_sems.at[my],
                device_id=(peer,),
                device_id_type=pl.DeviceIdType.MESH,
            )
            rdma.start()
            sends.append(rdma)
        x_copies[N_DEV - 1].wait()
        recv_buf[my] = x_vmem[N_DEV - 1].astype(FP8)

        acc = jnp.zeros((m_loc, n), jnp.float32)
        for i in range(N_DEV):
            if i > 0:
                recv = pltpu.make_async_remote_copy(
                    src_ref=send_buf.at[0],
                    dst_ref=recv_buf.at[srcs[i]],
                    send_sem=send_sems.at[0],
                    recv_sem=recv_sems.at[srcs[i]],
                    device_id=(srcs[i],),
                    device_id_type=pl.DeviceIdType.MESH,
                )
                recv.wait_recv()
            w_copies[i].wait()
            a_i = recv_buf[srcs[i]].astype(jnp.float32)
            acc = acc + jnp.dot(
                a_i, w_vmem[i % W_SLOTS],
                preferred_element_type=jnp.float32,
            )
            if i + W_SLOTS < N_DEV:
                w_copies[i + W_SLOTS] = start_w(i + W_SLOTS)

        scale = sx_ref[0] * sw_ref[0]
        half = n // 2
        out_copies = []
        for h in range(2):
            y = acc[:, h * half:(h + 1) * half] * scale
            y_vmem[h] = y * jax.nn.sigmoid(y)
            cp = pltpu.make_async_copy(
                y_vmem.at[h],
                out_hbm.at[:, pl.ds(h * half, half)],
                out_sems.at[h],
            )
            cp.start()
            out_copies.append(cp)
        for cp in out_copies:
            cp.wait()

        for rdma in sends:
            rdma.wait_send()

    return pl.pallas_call(
        body,
        out_shape=jax.ShapeDtypeStruct((m_loc, n), jnp.float32),
        in_specs=[
            pl.BlockSpec(memory_space=pl.ANY),
            pl.BlockSpec(memory_space=pl.ANY),
            pl.BlockSpec(memory_space=pltpu.SMEM),
            pl.BlockSpec(memory_space=pltpu.SMEM),
        ],
        out_specs=pl.BlockSpec(memory_space=pl.ANY),
        scratch_shapes=[
            pltpu.VMEM((N_DEV, m_loc, k_loc), jnp.float32),
            pltpu.VMEM((W_SLOTS, k_loc, n), jnp.float32),
            pltpu.VMEM((N_DEV - 1, m_loc, k_loc), FP8),
            pltpu.VMEM((N_DEV, m_loc, k_loc), FP8),
            pltpu.VMEM((2, m_loc, n // 2), jnp.float32),
            pltpu.SemaphoreType.DMA((N_DEV,)),
            pltpu.SemaphoreType.DMA((W_SLOTS,)),
            pltpu.SemaphoreType.DMA((N_DEV - 1,)),
            pltpu.SemaphoreType.DMA((N_DEV,)),
            pltpu.SemaphoreType.DMA((2,)),
        ],
        compiler_params=pltpu.CompilerParams(
            collective_id=0,
            vmem_limit_bytes=63 * 1024 * 1024,
        ),
    )(x, w_mat, scale_x, scale_w)


# device time: 30329 ns/iter; 1.6667x vs baseline; 1.0022x over previous
import jax
import jax.numpy as jnp
from jax import lax
from jax.experimental import pallas as pl
from jax.experimental.pallas import tpu as pltpu

N_DEV = 8
FP8 = jnp.float8_e4m3fn
W_SLOTS = 4


def kernel(x, w_mat, scale_x, scale_w):
    m_glob, k_loc = x.shape
    k_glob, n = w_mat.shape
    m_loc = m_glob // N_DEV

    def body(x_hbm, w_hbm, sx_ref, sw_ref, out_hbm,
             x_vmem, w_vmem, send_buf, recv_buf, y_vmem,
             x_sems, w_sems, send_sems, recv_sems, out_sems):
        my = lax.axis_index("i")
        send_peers = [lax.rem(my + k, N_DEV) for k in range(1, N_DEV)]
        srcs = [my] + [lax.rem(my + N_DEV - i, N_DEV) for i in range(1, N_DEV)]

        x_order = send_peers + [my]
        x_copies = []
        for i, blk in enumerate(x_order):
            cp = pltpu.make_async_copy(
                x_hbm.at[pl.ds(blk * m_loc, m_loc), :],
                x_vmem.at[i],
                x_sems.at[i],
            )
            cp.start()
            x_copies.append(cp)

        def start_w(i):
            cp = pltpu.make_async_copy(
                w_hbm.at[pl.ds(srcs[i] * k_loc, k_loc), :],
                w_vmem.at[i % W_SLOTS],
                w_sems.at[i % W_SLOTS],
            )
            cp.start()
            return cp

        w_copies = {i: start_w(i) for i in range(W_SLOTS)}

        barrier = pltpu.get_barrier_semaphore()
        for peer in send_peers:
            pl.semaphore_signal(
                barrier, inc=1,
                device_id=(peer,), device_id_type=pl.DeviceIdType.MESH,
            )
        pl.semaphore_wait(barrier, N_DEV - 1)

        sends = []
        for k in range(1, N_DEV):
            peer = send_peers[k - 1]
            x_copies[k - 1].wait()
            send_buf[k - 1] = x_vmem[k - 1].astype(FP8)
            rdma = pltpu.make_async_remote_copy(
                src_ref=send_buf.at[k - 1],
                dst_ref=recv_buf.at[my],
                send_sem=send_sems.at[k - 1],
                recv_sem=recv_sems.at[my],
                device_id=(peer,),
                device_id_type=pl.DeviceIdType.MESH,
            )
            rdma.start()
            sends.append(rdma)
        x_copies[N_DEV - 1].wait()
        recv_buf[my] = x_vmem[N_DEV - 1].astype(FP8)

        scale = sx_ref[0] * sw_ref[0]
        half = n // 2
        acc = jnp.zeros((m_loc, n), jnp.float32)
        for i in range(N_DEV - 1):
            if i > 0:
                recv = pltpu.make_async_remote_copy(
                    src_ref=send_buf.at[0],
                    dst_ref=recv_buf.at[srcs[i]],
                    send_sem=send_sems.at[0],
                    recv_sem=recv_sems.at[srcs[i]],
                    device_id=(srcs[i],),
                    device_id_type=pl.DeviceIdType.MESH,
                )
                recv.wait_recv()
            w_copies[i].wait()
            a_i = recv_buf[srcs[i]].astype(jnp.float32)
            acc = acc + jnp.dot(
                a_i, w_vmem[i % W_SLOTS],
                preferred_element_type=jnp.float32,
            )
            if i + W_SLOTS < N_DEV:
                w_copies[i + W_SLOTS] = start_w(i + W_SLOTS)

        last = N_DEV - 1
        recv = pltpu.make_async_remote_copy(
            src_ref=send_buf.at[0],
            dst_ref=recv_buf.at[srcs[last]],
            send_sem=send_sems.at[0],
            recv_sem=recv_sems.at[srcs[last]],
            device_id=(srcs[last],),
            device_id_type=pl.DeviceIdType.MESH,
        )
        recv.wait_recv()
        w_copies[last].wait()
        a_last = recv_buf[srcs[last]].astype(jnp.float32)
        out_copies = []
        for h in range(2):
            cols = pl.ds(h * half, half)
            y = acc[:, h * half:(h + 1) * half] + jnp.dot(
                a_last, w_vmem[last % W_SLOTS][:, h * half:(h + 1) * half],
                preferred_element_type=jnp.float32,
            )
            y = y * scale
            y_vmem[h] = y * jax.nn.sigmoid(y)
            cp = pltpu.make_async_copy(
                y_vmem.at[h], out_hbm.at[:, cols], out_sems.at[h],
            )
            cp.start()
            out_copies.append(cp)
        for cp in out_copies:
            cp.wait()

        for rdma in sends:
            rdma.wait_send()

    return pl.pallas_call(
        body,
        out_shape=jax.ShapeDtypeStruct((m_loc, n), jnp.float32),
        in_specs=[
            pl.BlockSpec(memory_space=pl.ANY),
            pl.BlockSpec(memory_space=pl.ANY),
            pl.BlockSpec(memory_space=pltpu.SMEM),
            pl.BlockSpec(memory_space=pltpu.SMEM),
        ],
        out_specs=pl.BlockSpec(memory_space=pl.ANY),
        scratch_shapes=[
            pltpu.VMEM((N_DEV, m_loc, k_loc), jnp.float32),
            pltpu.VMEM((W_SLOTS, k_loc, n), jnp.float32),
            pltpu.VMEM((N_DEV - 1, m_loc, k_loc), FP8),
            pltpu.VMEM((N_DEV, m_loc, k_loc), FP8),
            pltpu.VMEM((2, m_loc, n // 2), jnp.float32),
            pltpu.SemaphoreType.DMA((N_DEV,)),
            pltpu.SemaphoreType.DMA((W_SLOTS,)),
            pltpu.SemaphoreType.DMA((N_DEV - 1,)),
            pltpu.SemaphoreType.DMA((N_DEV,)),
            pltpu.SemaphoreType.DMA((2,)),
        ],
        compiler_params=pltpu.CompilerParams(
            collective_id=0,
            vmem_limit_bytes=63 * 1024 * 1024,
        ),
    )(x, w_mat, scale_x, scale_w)
